# baseline (speedup 1.0000x reference)
import functools
import numpy as np
import jax
import jax.numpy as jnp
from jax import lax
from jax.experimental import pallas as pl
from jax.experimental.pallas import tpu as pltpu
from jax.experimental.pallas import tpu_sc as plsc

DIM = 32
SCALE = np.sqrt(np.float32(DIM)).astype(np.float32)
NC, NS = 2, 16
NW = NC * NS
NB2 = 2
NBUF = 4
GATHER_AHEAD = 3


@functools.lru_cache(maxsize=None)
def _make(B1, B2):
    cols_w = B1 // NW
    n_chunks = B2 // NB2
    n_groups = n_chunks // NBUF
    assert B2 % NB2 == 0 and n_chunks % NBUF == 0
    mesh = plsc.VectorSubcoreMesh(
        core_axis_name="c", subcore_axis_name="s",
        num_cores=NC, num_subcores=NS)

    @functools.partial(
        pl.kernel,
        out_type=jax.ShapeDtypeStruct((B1, B2, DIM), jnp.float32),
        mesh=mesh,
        scratch_types=(
            [pltpu.VMEM((B2, cols_w), jnp.int32)]
            + [pltpu.VMEM((NB2 * cols_w, DIM), jnp.float32)] * NBUF
            + [pltpu.VMEM((cols_w, NB2, DIM), jnp.float32)] * NBUF
            + [pltpu.SemaphoreType.DMA] * (2 * NBUF)
        ),
        compiler_params=pltpu.CompilerParams(use_tc_tiling_on_sc=False),
    )
    def emb_kernel(table_hbm, xt_hbm, out_hbm, idx_v, *scratch):
        gbufs = scratch[:NBUF]
        obufs = scratch[NBUF:2 * NBUF]
        gsems = scratch[2 * NBUF:3 * NBUF]
        ssems = scratch[3 * NBUF:]
        wid = lax.axis_index("s") * NC + lax.axis_index("c")
        col0 = wid * cols_w

        def fire_gather(c, b):
            for s in range(NB2):
                pltpu.async_copy(
                    table_hbm.at[idx_v.at[c * NB2 + s]],
                    gbufs[b].at[pl.ds(s * cols_w, cols_w)],
                    gsems[b])

        def wait_gather(b):
            pltpu.make_async_copy(
                table_hbm.at[pl.ds(0, NB2 * cols_w)],
                gbufs[b], gsems[b]).wait()

        def fire_scatter(c, b):
            pltpu.async_copy(
                obufs[b],
                out_hbm.at[pl.ds(col0, cols_w), pl.ds(c * NB2, NB2)],
                ssems[b])

        def wait_scatter(b):
            pltpu.make_async_copy(
                obufs[b],
                out_hbm.at[pl.ds(0, cols_w), pl.ds(0, NB2)],
                ssems[b]).wait()

        def scale(b):
            gbuf, obuf = gbufs[b], obufs[b]

            @plsc.parallel_loop(0, cols_w, step=1, unroll=4)
            def _scale(r):
                for s in range(NB2):
                    lo = gbuf[s * cols_w + r, pl.ds(0, 16)]
                    hi = gbuf[s * cols_w + r, pl.ds(16, 16)]
                    obuf[r, s, pl.ds(0, 16)] = lo * SCALE
                    obuf[r, s, pl.ds(16, 16)] = hi * SCALE

        pltpu.sync_copy(xt_hbm.at[:, pl.ds(col0, cols_w)], idx_v)

        for c in range(GATHER_AHEAD):
            fire_gather(c, c % NBUF)

        @pl.loop(0, n_groups)
        def _group(g):
            for i in range(NBUF):
                c = g * NBUF + i
                wait_gather(i)

                @pl.when(c >= NBUF)
                def _():
                    wait_scatter(i)

                scale(i)
                fire_scatter(c, i)

                @pl.when(c + GATHER_AHEAD < n_chunks)
                def _():
                    fire_gather(c + GATHER_AHEAD, (i + GATHER_AHEAD) % NBUF)

        for c in range(n_chunks - NBUF, n_chunks):
            wait_scatter(c % NBUF)

    return emb_kernel


def kernel(x, table):
    B1, B2 = x.shape
    xt = jnp.transpose(x, (1, 0)).astype(jnp.int32)
    return _make(B1, B2)(table, xt)


# speedup vs baseline: 1.0491x; 1.0491x over previous
import functools
import numpy as np
import jax
import jax.numpy as jnp
from jax import lax
from jax.experimental import pallas as pl
from jax.experimental.pallas import tpu as pltpu
from jax.experimental.pallas import tpu_sc as plsc

DIM = 32
SCALE = np.sqrt(np.float32(DIM)).astype(np.float32)
NC, NS = 2, 16
NW = NC * NS
NB2 = 2
NBUF = 4
GATHER_AHEAD = 3


@functools.lru_cache(maxsize=None)
def _make(B1, B2):
    cols_w = B1 // NW
    n_chunks = B2 // NB2
    n_groups = n_chunks // NBUF
    assert B2 % NB2 == 0 and n_chunks % NBUF == 0
    mesh = plsc.VectorSubcoreMesh(
        core_axis_name="c", subcore_axis_name="s",
        num_cores=NC, num_subcores=NS)

    @functools.partial(
        pl.kernel,
        out_type=jax.ShapeDtypeStruct((B2, DIM, B1), jnp.float32),
        mesh=mesh,
        scratch_types=(
            [pltpu.VMEM((B2, cols_w), jnp.int32)]
            + [pltpu.VMEM((NB2 * cols_w, DIM), jnp.float32)] * NBUF
            + [pltpu.VMEM((NB2, DIM, cols_w), jnp.float32)] * NBUF
            + [pltpu.SemaphoreType.DMA] * (2 * NBUF)
        ),
        compiler_params=pltpu.CompilerParams(
            use_tc_tiling_on_sc=False, needs_layout_passes=False),
    )
    def emb_kernel(table_hbm, xt_hbm, out_hbm, idx_v, *scratch):
        gbufs = scratch[:NBUF]
        obufs = scratch[NBUF:2 * NBUF]
        gsems = scratch[2 * NBUF:3 * NBUF]
        ssems = scratch[3 * NBUF:]
        wid = lax.axis_index("s") * NC + lax.axis_index("c")
        col0 = wid * cols_w

        def fire_gather(c, b):
            for s in range(NB2):
                pltpu.async_copy(
                    table_hbm.at[idx_v.at[c * NB2 + s]],
                    gbufs[b].at[pl.ds(s * cols_w, cols_w)],
                    gsems[b])

        def wait_gather(b):
            pltpu.make_async_copy(
                table_hbm.at[pl.ds(0, NB2 * cols_w)],
                gbufs[b], gsems[b]).wait()

        def fire_scatter(c, b):
            pltpu.async_copy(
                obufs[b],
                out_hbm.at[pl.ds(c * NB2, NB2), :, pl.ds(col0, cols_w)],
                ssems[b])

        def wait_scatter(b):
            pltpu.make_async_copy(
                obufs[b],
                out_hbm.at[pl.ds(0, NB2), :, pl.ds(0, cols_w)],
                ssems[b]).wait()

        lane = lax.iota(jnp.int32, 16)
        row_ids = [[lane + (s * cols_w + 16 * g)
                    for g in range(cols_w // 16)] for s in range(NB2)]

        def scale(b):
            gbuf, obuf = gbufs[b], obufs[b]

            @plsc.parallel_loop(0, DIM, step=1, unroll=2)
            def _scale(d):
                dcol = jnp.broadcast_to(d, (16,))
                for s in range(NB2):
                    for g in range(cols_w // 16):
                        vec = plsc.load_gather(gbuf, [row_ids[s][g], dcol])
                        obuf[s, d, pl.ds(16 * g, 16)] = vec * SCALE

        pltpu.sync_copy(xt_hbm.at[:, pl.ds(col0, cols_w)], idx_v)

        for c in range(GATHER_AHEAD):
            fire_gather(c, c % NBUF)

        @pl.loop(0, n_groups)
        def _group(g):
            for i in range(NBUF):
                c = g * NBUF + i
                wait_gather(i)

                @pl.when(c >= NBUF)
                def _():
                    wait_scatter(i)

                scale(i)
                fire_scatter(c, i)

                @pl.when(c + GATHER_AHEAD < n_chunks)
                def _():
                    fire_gather(c + GATHER_AHEAD, (i + GATHER_AHEAD) % NBUF)

        for c in range(n_chunks - NBUF, n_chunks):
            wait_scatter(c % NBUF)

    return emb_kernel


def kernel(x, table):
    B1, B2 = x.shape
    xt = jnp.transpose(x, (1, 0)).astype(jnp.int32)
    out = _make(B1, B2)(table, xt)
    return jnp.transpose(out, (2, 0, 1))


# speedup vs baseline: 1.4353x; 1.3682x over previous
import functools
import numpy as np
import jax
import jax.numpy as jnp
from jax import lax
from jax.experimental import pallas as pl
from jax.experimental.pallas import tpu as pltpu
from jax.experimental.pallas import tpu_sc as plsc

DIM = 32
SCALE = np.sqrt(np.float32(DIM)).astype(np.float32)
NC, NS = 2, 16
NW = NC * NS
NB2 = 2
NBUF = 4
GATHER_AHEAD = 3


@functools.lru_cache(maxsize=None)
def _make(B1, B2):
    cols_w = B1 // NW
    n_chunks = B2 // NB2
    n_groups = n_chunks // NBUF
    assert B2 % NB2 == 0 and n_chunks % NBUF == 0
    mesh = plsc.VectorSubcoreMesh(
        core_axis_name="c", subcore_axis_name="s",
        num_cores=NC, num_subcores=NS)

    @functools.partial(
        pl.kernel,
        out_type=jax.ShapeDtypeStruct((B2, DIM, B1), jnp.float32),
        mesh=mesh,
        scratch_types=(
            [pltpu.VMEM((B2, cols_w), jnp.int32)]
            + [pltpu.VMEM((NB2 * cols_w, DIM), jnp.float32)] * NBUF
            + [pltpu.VMEM((NB2, DIM, cols_w + 1), jnp.float32)] * NBUF
            + [pltpu.SemaphoreType.DMA] * (2 * NBUF)
        ),
        compiler_params=pltpu.CompilerParams(
            use_tc_tiling_on_sc=False, needs_layout_passes=False),
    )
    def emb_kernel(table_hbm, xt_hbm, out_hbm, idx_v, *scratch):
        gbufs = scratch[:NBUF]
        obufs = scratch[NBUF:2 * NBUF]
        gsems = scratch[2 * NBUF:3 * NBUF]
        ssems = scratch[3 * NBUF:]
        wid = lax.axis_index("s") * NC + lax.axis_index("c")
        col0 = wid * cols_w

        def fire_gather(c, b):
            for s in range(NB2):
                pltpu.async_copy(
                    table_hbm.at[idx_v.at[c * NB2 + s]],
                    gbufs[b].at[pl.ds(s * cols_w, cols_w)],
                    gsems[b])

        def wait_gather(b):
            pltpu.make_async_copy(
                table_hbm.at[pl.ds(0, NB2 * cols_w)],
                gbufs[b], gsems[b]).wait()

        def fire_scatter(c, b):
            pltpu.async_copy(
                obufs[b].at[:, :, pl.ds(0, cols_w)],
                out_hbm.at[pl.ds(c * NB2, NB2), :, pl.ds(col0, cols_w)],
                ssems[b])

        def wait_scatter(b):
            pltpu.make_async_copy(
                obufs[b].at[:, :, pl.ds(0, cols_w)],
                out_hbm.at[pl.ds(0, NB2), :, pl.ds(0, cols_w)],
                ssems[b]).wait()

        lane = lax.iota(jnp.int32, 16)
        dim_rows = [lane + 16 * h for h in range(DIM // 16)]
        s_ids = [jnp.broadcast_to(jnp.int32(s), (16,)) for s in range(NB2)]

        def scale(b):
            gbuf, obuf = gbufs[b], obufs[b]

            @plsc.parallel_loop(0, cols_w, step=1, unroll=4)
            def _scale(l):
                lcol = jnp.broadcast_to(l, (16,))
                for s in range(NB2):
                    for h in range(DIM // 16):
                        vec = gbuf[s * cols_w + l, pl.ds(16 * h, 16)]
                        plsc.store_scatter(
                            obuf, [s_ids[s], dim_rows[h], lcol], vec * SCALE)

        pltpu.sync_copy(xt_hbm.at[:, pl.ds(col0, cols_w)], idx_v)

        for c in range(GATHER_AHEAD):
            fire_gather(c, c % NBUF)

        @pl.loop(0, n_groups)
        def _group(g):
            for i in range(NBUF):
                c = g * NBUF + i
                wait_gather(i)

                @pl.when(c >= NBUF)
                def _():
                    wait_scatter(i)

                scale(i)
                fire_scatter(c, i)

                @pl.when(c + GATHER_AHEAD < n_chunks)
                def _():
                    fire_gather(c + GATHER_AHEAD, (i + GATHER_AHEAD) % NBUF)

        for c in range(n_chunks - NBUF, n_chunks):
            wait_scatter(c % NBUF)

    return emb_kernel


def kernel(x, table):
    B1, B2 = x.shape
    xt = jnp.transpose(x, (1, 0)).astype(jnp.int32)
    out = _make(B1, B2)(table, xt)
    return jnp.transpose(out, (2, 0, 1))
